# baseline (speedup 1.0000x reference)
import functools
import math

import numpy as np
import jax
import jax.numpy as jnp
from jax import lax
from jax.experimental import pallas as pl
from jax.experimental.pallas import tpu as pltpu
from jax.experimental.pallas import tpu_sc as plsc

D_MODEL = 128
MAX_SEQ = 8192
NC, NS = 2, 16
NW = NC * NS
LANES = 16
R = 128
SCALE = math.sqrt(float(D_MODEL))


def _make_pe_np(max_seq, d_model):
    position = np.arange(max_seq, dtype=np.float32)[:, None]
    div_term = np.exp(
        np.arange(0, d_model, 2, dtype=np.float32) * (-math.log(10000.0) / d_model))
    pe = np.zeros((max_seq, d_model), dtype=np.float32)
    pe[:, 0::2] = np.sin(position * div_term)
    pe[:, 1::2] = np.cos(position * div_term)
    return pe


@functools.cache
def _build(batch, seq_len, d):
    assert seq_len % NW == 0
    spw = seq_len // NW
    assert spw % R == 0
    hpw = spw // R
    nch = batch * hpw
    mesh = plsc.VectorSubcoreMesh(core_axis_name="c", subcore_axis_name="s")

    @functools.partial(
        pl.kernel,
        mesh=mesh,
        out_type=jax.ShapeDtypeStruct((batch, seq_len, d), jnp.float32),
        scratch_types=[
            pltpu.VMEM((batch, spw), jnp.int32),
            pltpu.VMEM((3, R, d), jnp.float32),
            pltpu.VMEM((spw, d), jnp.float32),
            pltpu.SemaphoreType.DMA,
            pltpu.SemaphoreType.DMA,
            pltpu.SemaphoreType.DMA,
            pltpu.SemaphoreType.DMA,
            pltpu.SemaphoreType.DMA,
            pltpu.SemaphoreType.DMA,
            pltpu.SemaphoreType.DMA,
        ],
    )
    def emb_kernel(table_hbm, x_hbm, pe_hbm, out_hbm,
                   idx_v, rows_v, pe_v, g0, g1, g2, o0, o1, o2, psem):
        gsem = (g0, g1, g2)
        osem = (o0, o1, o2)
        wid = lax.axis_index("s") * NC + lax.axis_index("c")
        s0 = wid * spw

        pdma = pltpu.async_copy(pe_hbm.at[pl.ds(s0, spw)], pe_v, psem)
        for b in range(batch):
            pltpu.sync_copy(x_hbm.at[b, pl.ds(s0, spw)], idx_v.at[b])

        chunks = [(b, h) for b in range(batch) for h in range(hpw)]

        def gather(c, buf):
            b, h = chunks[c]
            return pltpu.async_copy(
                table_hbm.at[idx_v.at[b, pl.ds(h * R, R)]], rows_v.at[buf],
                gsem[buf])

        gd = [None] * nch
        od = [None] * nch
        gd[0] = gather(0, 0)
        gd[1] = gather(1, 1)
        pdma.wait()
        for c in range(nch):
            b, h = chunks[c]
            bb = c % 3
            if c + 2 < nch:
                nb = (c + 2) % 3
                if c >= 1:
                    od[c - 1].wait()
                gd[c + 2] = gather(c + 2, nb)
            gd[c].wait()

            def comp(i, carry, _bb=bb, _h=h):
                for j in range(d // LANES):
                    sl = pl.ds(j * LANES, LANES)
                    rows_v[_bb, i, sl] = (
                        rows_v[_bb, i, sl] * SCALE + pe_v[_h * R + i, sl])
                return carry

            lax.fori_loop(0, R, comp, 0, unroll=2)
            od[c] = pltpu.async_copy(
                rows_v.at[bb], out_hbm.at[b, pl.ds(s0 + h * R, R)], osem[bb])
        for c in (nch - 3, nch - 2, nch - 1):
            if c >= 0:
                od[c].wait()

    return emb_kernel


def kernel(x, table):
    batch, seq_len = x.shape
    d = table.shape[1]
    pe = jnp.asarray(_make_pe_np(MAX_SEQ, d)[:seq_len])
    return _build(batch, seq_len, d)(table, x.astype(jnp.int32), pe)


# speedup vs baseline: 1.2589x; 1.2589x over previous
import functools
import math

import numpy as np
import jax
import jax.numpy as jnp
from jax import lax
from jax.experimental import pallas as pl
from jax.experimental.pallas import tpu as pltpu
from jax.experimental.pallas import tpu_sc as plsc

D_MODEL = 128
MAX_SEQ = 8192
NC, NS = 2, 16
NW = NC * NS
LANES = 16
R = 128
SCALE = math.sqrt(float(D_MODEL))


def _make_pe_np(max_seq, d_model):
    position = np.arange(max_seq, dtype=np.float32)[:, None]
    div_term = np.exp(
        np.arange(0, d_model, 2, dtype=np.float32) * (-math.log(10000.0) / d_model))
    pe = np.zeros((max_seq, d_model), dtype=np.float32)
    pe[:, 0::2] = np.sin(position * div_term)
    pe[:, 1::2] = np.cos(position * div_term)
    return pe


@functools.cache
def _build(batch, seq_len, d):
    assert seq_len % NW == 0
    spw = seq_len // NW
    assert spw % R == 0
    hpw = spw // R
    nch = batch * hpw
    mesh = plsc.VectorSubcoreMesh(core_axis_name="c", subcore_axis_name="s")

    @functools.partial(
        pl.kernel,
        mesh=mesh,
        out_type=jax.ShapeDtypeStruct((batch, seq_len, d), jnp.float32),
        scratch_types=[
            pltpu.VMEM((batch, spw), jnp.int32),
            pltpu.VMEM((3, R, d), jnp.float32),
            pltpu.VMEM((spw, d), jnp.float32),
            pltpu.SemaphoreType.DMA,
            pltpu.SemaphoreType.DMA,
            pltpu.SemaphoreType.DMA,
            pltpu.SemaphoreType.DMA,
            pltpu.SemaphoreType.DMA,
            pltpu.SemaphoreType.DMA,
            pltpu.SemaphoreType.DMA,
            pltpu.SemaphoreType.DMA,
            pltpu.SemaphoreType.DMA,
        ],
    )
    def emb_kernel(table_hbm, x_hbm, pe_hbm, out_hbm,
                   idx_v, rows_v, pe_v, g0, g1, g2, o0, o1, o2, psem, i0sem, isem):
        gsem = (g0, g1, g2)
        osem = (o0, o1, o2)
        wid = lax.axis_index("s") * NC + lax.axis_index("c")
        s0 = wid * spw

        pdma = pltpu.async_copy(pe_hbm.at[pl.ds(s0, spw)], pe_v, psem)
        idma = [pltpu.async_copy(x_hbm.at[b, pl.ds(s0, spw)], idx_v.at[b],
                                 i0sem if b == 0 else isem)
                for b in range(batch)]

        chunks = [(b, h) for b in range(batch) for h in range(hpw)]

        def gather(c, buf):
            b, h = chunks[c]
            return pltpu.async_copy(
                table_hbm.at[idx_v.at[b, pl.ds(h * R, R)]], rows_v.at[buf],
                gsem[buf])

        gd = [None] * nch
        od = [None] * nch
        idma[0].wait()
        gd[0] = gather(0, 0)
        gd[1] = gather(1, 1)
        for c in idma[1:]:
            c.wait()
        pdma.wait()
        for c in range(nch):
            b, h = chunks[c]
            bb = c % 3
            if c + 2 < nch:
                nb = (c + 2) % 3
                if c >= 1:
                    od[c - 1].wait()
                gd[c + 2] = gather(c + 2, nb)
            gd[c].wait()

            def comp(i, carry, _bb=bb, _h=h):
                for j in range(d // LANES):
                    sl = pl.ds(j * LANES, LANES)
                    rows_v[_bb, i, sl] = (
                        rows_v[_bb, i, sl] * SCALE + pe_v[_h * R + i, sl])
                return carry

            lax.fori_loop(0, R, comp, 0)
            od[c] = pltpu.async_copy(
                rows_v.at[bb], out_hbm.at[b, pl.ds(s0 + h * R, R)], osem[bb])
        for c in (nch - 3, nch - 2, nch - 1):
            if c >= 0:
                od[c].wait()

    return emb_kernel


def kernel(x, table):
    batch, seq_len = x.shape
    d = table.shape[1]
    pe = jnp.asarray(_make_pe_np(MAX_SEQ, d)[:seq_len])
    return _build(batch, seq_len, d)(table, x.astype(jnp.int32), pe)
